# Initial kernel scaffold; baseline (speedup 1.0000x reference)
import jax
import jax.numpy as jnp
from jax.experimental import pallas as pl


def kernel(features, member_nodes, segment_ids):
    raise NotImplementedError("write your pallas kernel here")


# speedup vs baseline: 2.6417x; 2.6417x over previous
import functools

import jax
import jax.numpy as jnp
from jax import lax
from jax.experimental import pallas as pl
from jax.experimental.pallas import tpu as pltpu
from jax.experimental.pallas import tpu_sc as plsc

H = 10000
D = 128
NW = 32
HPW = -(-H // NW)
C = 128
ACC = (HPW + 1) * D
SENTINEL = 1 << 30


def _tile_body(feat_hbm, mn_hbm, seg_hbm, meta_hbm, out_hbm,
               meta_v, mn_v, seg_v, rows_v, acc_v, sem):
    w = lax.axis_index("s") * 2 + lax.axis_index("c")
    pltpu.sync_copy(meta_hbm.at[pl.ds(pl.multiple_of(w * 16, 8), 16)], meta_v)
    mv = meta_v[pl.ds(0, 16)]
    start8 = mv[0]
    nch = mv[1]
    h0 = w * HPW

    neg = jnp.full((16,), -jnp.inf, dtype=jnp.float32)

    @pl.loop(0, ACC, step=16)
    def _init(i):
        acc_v[pl.ds(i, 16)] = neg

    @pl.loop(0, nch)
    def _chunk(cch):
        off = pl.multiple_of(start8 + cch * C, 8)
        pltpu.sync_copy(mn_hbm.at[pl.ds(off, C)], mn_v)
        pltpu.sync_copy(seg_hbm.at[pl.ds(off, C)], seg_v)
        pltpu.async_copy(feat_hbm.at[mn_v], rows_v, sem).wait()

        @pl.loop(0, C, step=16)
        def _grp(g):
            sv = seg_v[pl.ds(g, 16)]
            for i in range(16):
                ls = sv[i] - h0
                ls = lax.select((ls < 0) | (ls >= HPW), HPW, ls)
                a0 = ls * D
                for k in range(D // 16):
                    sl = pl.ds(a0 + k * 16, 16)
                    acc_v[sl] = jnp.maximum(
                        acc_v[sl], rows_v[g + i, pl.ds(k * 16, 16)])

    pltpu.sync_copy(acc_v.at[pl.ds(0, HPW * D)],
                    out_hbm.at[pl.ds(pl.multiple_of(w * HPW * D, 8), HPW * D)])


def kernel(features, member_nodes, segment_ids):
    e = member_nodes.shape[0]
    cuts = jnp.arange(NW + 1, dtype=jnp.int32) * HPW
    bounds = jnp.searchsorted(segment_ids, cuts).astype(jnp.int32)
    starts8 = (bounds[:NW] // 8) * 8
    nch = (bounds[1:] - starts8 + (C - 1)) // C
    meta = jnp.zeros((NW, 16), jnp.int32)
    meta = meta.at[:, 0].set(starts8).at[:, 1].set(nch).reshape(-1)
    pad = C + 8
    mn_p = jnp.concatenate([member_nodes, jnp.zeros((pad,), jnp.int32)])
    seg_p = jnp.concatenate([segment_ids, jnp.full((pad,), SENTINEL, jnp.int32)])

    mesh = plsc.VectorSubcoreMesh(core_axis_name="c", subcore_axis_name="s")
    run = pl.kernel(
        _tile_body,
        out_type=jax.ShapeDtypeStruct((NW * HPW * D,), jnp.float32),
        mesh=mesh,
        scratch_types=[
            pltpu.VMEM((16,), jnp.int32),
            pltpu.VMEM((C,), jnp.int32),
            pltpu.VMEM((C,), jnp.int32),
            pltpu.VMEM((C, D), jnp.float32),
            pltpu.VMEM((ACC,), jnp.float32),
            pltpu.SemaphoreType.DMA,
        ],
    )
    out = run(features, mn_p, seg_p, meta)
    return out.reshape(NW * HPW, D)[:H]
